# baseline (speedup 1.0000x reference)
import jax
import jax.numpy as jnp
from jax.experimental import pallas as pl
from jax.experimental.pallas import tpu as pltpu

_C = 8192
_R = 8


def _body(x_sref, *refs):
    w_refs = refs[:_R]
    y_ref = refs[_R]
    logits_ref = refs[_R + 1]
    loss_ref = refs[_R + 2]
    acc_ref = refs[_R + 3]

    i = pl.program_id(0)

    rows = jnp.concatenate(
        [w_refs[j][...].reshape(1, _C) for j in range(_R)], axis=0)
    logits_ref[...] = rows

    m = jnp.max(rows, axis=1, keepdims=True)
    s = jnp.sum(jnp.exp(rows - m), axis=1, keepdims=True)
    lse = m + jnp.log(s)

    yv = y_ref[0, 0, :].reshape(_R, 1)
    col = jax.lax.broadcasted_iota(jnp.int32, (_R, _C), 1)
    picked = jnp.sum(jnp.where(col == yv, rows, 0.0), axis=1, keepdims=True)

    contrib = jnp.sum(lse - picked)

    @pl.when(i == 0)
    def _():
        acc_ref[0] = 0.0

    acc_ref[0] += contrib

    @pl.when(i == pl.num_programs(0) - 1)
    def _():
        loss_ref[...] = jnp.full((1, 1), acc_ref[0], jnp.float32)


def kernel(x, y, W):
    n_tok = x.size
    steps = n_tok // _R
    xf = x.reshape(-1).astype(jnp.int32)
    y3 = y.reshape(steps, 1, _R).astype(jnp.int32)
    W3 = W.reshape(W.shape[0], 1, _C)

    def w_spec(j):
        return pl.BlockSpec((1, 1, _C), lambda i, xs, j=j: (xs[i * _R + j], 0, 0))

    grid_spec = pltpu.PrefetchScalarGridSpec(
        num_scalar_prefetch=1,
        grid=(steps,),
        in_specs=[w_spec(j) for j in range(_R)] + [
            pl.BlockSpec((1, 1, _R), lambda i, xs: (i, 0, 0)),
        ],
        out_specs=[
            pl.BlockSpec((_R, _C), lambda i, xs: (i, 0)),
            pl.BlockSpec((1, 1), lambda i, xs: (0, 0)),
        ],
        scratch_shapes=[pltpu.SMEM((1,), jnp.float32)],
    )

    logits, loss = pl.pallas_call(
        _body,
        grid_spec=grid_spec,
        out_shape=[
            jax.ShapeDtypeStruct((n_tok, _C), jnp.float32),
            jax.ShapeDtypeStruct((1, 1), jnp.float32),
        ],
    )(xf, *([W3] * _R), y3)

    return (logits, (loss[0, 0] / n_tok).astype(jnp.float32))


# speedup vs baseline: 1.3603x; 1.3603x over previous
import jax
import jax.numpy as jnp
from jax.experimental import pallas as pl
from jax.experimental.pallas import tpu as pltpu

_C = 8192
_R = 16


def _body(x_sref, *refs):
    w_refs = refs[:_R]
    y_ref = refs[_R]
    logits_ref = refs[_R + 1]
    loss_ref = refs[_R + 2]
    acc_ref = refs[_R + 3]

    i = pl.program_id(0)

    rows = jnp.concatenate(
        [w_refs[j][...].reshape(1, _C) for j in range(_R)], axis=0)
    logits_ref[...] = rows

    m = jnp.max(rows, axis=1, keepdims=True)
    s = jnp.sum(jnp.exp(rows - m), axis=1, keepdims=True)
    lse = m + jnp.log(s)

    yv = y_ref[0, 0, :].reshape(_R, 1)
    col = jax.lax.broadcasted_iota(jnp.int32, (_R, _C), 1)
    picked = jnp.sum(jnp.where(col == yv, rows, 0.0), axis=1, keepdims=True)

    contrib = jnp.sum(lse - picked)

    @pl.when(i == 0)
    def _():
        acc_ref[0] = 0.0

    acc_ref[0] += contrib

    @pl.when(i == pl.num_programs(0) - 1)
    def _():
        loss_ref[...] = jnp.full((1, 1), acc_ref[0], jnp.float32)


def kernel(x, y, W):
    n_tok = x.size
    steps = n_tok // _R
    xf = x.reshape(-1).astype(jnp.int32)
    y3 = y.reshape(steps, 1, _R).astype(jnp.int32)
    W3 = W.reshape(W.shape[0], 1, _C)

    def w_spec(j):
        return pl.BlockSpec((1, 1, _C), lambda i, xs, j=j: (xs[i * _R + j], 0, 0))

    grid_spec = pltpu.PrefetchScalarGridSpec(
        num_scalar_prefetch=1,
        grid=(steps,),
        in_specs=[w_spec(j) for j in range(_R)] + [
            pl.BlockSpec((1, 1, _R), lambda i, xs: (i, 0, 0)),
        ],
        out_specs=[
            pl.BlockSpec((_R, _C), lambda i, xs: (i, 0)),
            pl.BlockSpec((1, 1), lambda i, xs: (0, 0)),
        ],
        scratch_shapes=[pltpu.SMEM((1,), jnp.float32)],
    )

    logits, loss = pl.pallas_call(
        _body,
        grid_spec=grid_spec,
        out_shape=[
            jax.ShapeDtypeStruct((n_tok, _C), jnp.float32),
            jax.ShapeDtypeStruct((1, 1), jnp.float32),
        ],
    )(xf, *([W3] * _R), y3)

    return (logits, (loss[0, 0] / n_tok).astype(jnp.float32))


# speedup vs baseline: 1.6630x; 1.2225x over previous
import jax
import jax.numpy as jnp
from jax.experimental import pallas as pl
from jax.experimental.pallas import tpu as pltpu

_C = 8192
_R = 32


def _body(x_sref, *refs):
    w_refs = refs[:_R]
    y_ref = refs[_R]
    logits_ref = refs[_R + 1]
    loss_ref = refs[_R + 2]
    acc_ref = refs[_R + 3]

    i = pl.program_id(0)

    rows = jnp.concatenate(
        [w_refs[j][...].reshape(1, _C) for j in range(_R)], axis=0)
    logits_ref[...] = rows

    m = jnp.max(rows, axis=1, keepdims=True)
    s = jnp.sum(jnp.exp(rows - m), axis=1, keepdims=True)
    lse = m + jnp.log(s)

    yv = y_ref[0, 0, :].reshape(_R, 1)
    col = jax.lax.broadcasted_iota(jnp.int32, (_R, _C), 1)
    picked = jnp.sum(jnp.where(col == yv, rows, 0.0), axis=1, keepdims=True)

    contrib = jnp.sum(lse - picked)

    @pl.when(i == 0)
    def _():
        acc_ref[0] = 0.0

    acc_ref[0] += contrib

    @pl.when(i == pl.num_programs(0) - 1)
    def _():
        loss_ref[...] = jnp.full((1, 1), acc_ref[0], jnp.float32)


def kernel(x, y, W):
    n_tok = x.size
    steps = n_tok // _R
    xf = x.reshape(-1).astype(jnp.int32)
    y3 = y.reshape(steps, 1, _R).astype(jnp.int32)
    W3 = W.reshape(W.shape[0], 1, _C)

    def w_spec(j):
        return pl.BlockSpec((1, 1, _C), lambda i, xs, j=j: (xs[i * _R + j], 0, 0))

    grid_spec = pltpu.PrefetchScalarGridSpec(
        num_scalar_prefetch=1,
        grid=(steps,),
        in_specs=[w_spec(j) for j in range(_R)] + [
            pl.BlockSpec((1, 1, _R), lambda i, xs: (i, 0, 0)),
        ],
        out_specs=[
            pl.BlockSpec((_R, _C), lambda i, xs: (i, 0)),
            pl.BlockSpec((1, 1), lambda i, xs: (0, 0)),
        ],
        scratch_shapes=[pltpu.SMEM((1,), jnp.float32)],
    )

    logits, loss = pl.pallas_call(
        _body,
        grid_spec=grid_spec,
        out_shape=[
            jax.ShapeDtypeStruct((n_tok, _C), jnp.float32),
            jax.ShapeDtypeStruct((1, 1), jnp.float32),
        ],
    )(xf, *([W3] * _R), y3)

    return (logits, (loss[0, 0] / n_tok).astype(jnp.float32))


# speedup vs baseline: 1.7561x; 1.0560x over previous
import jax
import jax.numpy as jnp
from jax.experimental import pallas as pl
from jax.experimental.pallas import tpu as pltpu

_C = 8192
_R = 64


def _body(x_sref, *refs):
    w_refs = refs[:_R]
    y_ref = refs[_R]
    logits_ref = refs[_R + 1]
    loss_ref = refs[_R + 2]
    acc_ref = refs[_R + 3]

    i = pl.program_id(0)

    rows = jnp.concatenate(
        [w_refs[j][...].reshape(1, _C) for j in range(_R)], axis=0)
    logits_ref[...] = rows

    m = jnp.max(rows, axis=1, keepdims=True)
    s = jnp.sum(jnp.exp(rows - m), axis=1, keepdims=True)
    lse = m + jnp.log(s)

    yv = y_ref[0, 0, :].reshape(_R, 1)
    col = jax.lax.broadcasted_iota(jnp.int32, (_R, _C), 1)
    picked = jnp.sum(jnp.where(col == yv, rows, 0.0), axis=1, keepdims=True)

    contrib = jnp.sum(lse - picked)

    @pl.when(i == 0)
    def _():
        acc_ref[0] = 0.0

    acc_ref[0] += contrib

    @pl.when(i == pl.num_programs(0) - 1)
    def _():
        loss_ref[...] = jnp.full((1, 1), acc_ref[0], jnp.float32)


def kernel(x, y, W):
    n_tok = x.size
    steps = n_tok // _R
    xf = x.reshape(-1).astype(jnp.int32)
    y3 = y.reshape(steps, 1, _R).astype(jnp.int32)
    W3 = W.reshape(W.shape[0], 1, _C)

    def w_spec(j):
        return pl.BlockSpec((1, 1, _C), lambda i, xs, j=j: (xs[i * _R + j], 0, 0))

    grid_spec = pltpu.PrefetchScalarGridSpec(
        num_scalar_prefetch=1,
        grid=(steps,),
        in_specs=[w_spec(j) for j in range(_R)] + [
            pl.BlockSpec((1, 1, _R), lambda i, xs: (i, 0, 0)),
        ],
        out_specs=[
            pl.BlockSpec((_R, _C), lambda i, xs: (i, 0)),
            pl.BlockSpec((1, 1), lambda i, xs: (0, 0)),
        ],
        scratch_shapes=[pltpu.SMEM((1,), jnp.float32)],
    )

    logits, loss = pl.pallas_call(
        _body,
        grid_spec=grid_spec,
        out_shape=[
            jax.ShapeDtypeStruct((n_tok, _C), jnp.float32),
            jax.ShapeDtypeStruct((1, 1), jnp.float32),
        ],
    )(xf, *([W3] * _R), y3)

    return (logits, (loss[0, 0] / n_tok).astype(jnp.float32))
